# baseline (speedup 1.0000x reference)
import functools
import jax
import jax.numpy as jnp
from jax import lax
from jax.experimental import pallas as pl
from jax.experimental.pallas import tpu as pltpu
from jax.experimental.pallas import tpu_sc as plsc

B, T = 8, 4096
V, D = 8192, 256
NW = 32
TOK_PER_W = (B * T) // NW
W = 64
PAIRS = TOK_PER_W // (2 * W)
L = 16
TILES_PER_B = T // TOK_PER_W
DBLK = D // 8
TBLK = T // 128
OSTRIDE = 129

_mesh = plsc.VectorSubcoreMesh(core_axis_name="c", subcore_axis_name="s")


@functools.partial(
    pl.kernel,
    mesh=_mesh,
    out_type=jax.ShapeDtypeStruct((B, DBLK, TBLK, 8, 128), jnp.float32),
    scratch_types=[
        pltpu.VMEM((8, 128), jnp.int32),
        pltpu.VMEM((TOK_PER_W,), jnp.int32),
        pltpu.VMEM((2, 2, W, 128), jnp.float32),
        pltpu.VMEM((2, DBLK, 8, OSTRIDE), jnp.float32),
        pltpu.SemaphoreType.DMA((2,)),
        pltpu.SemaphoreType.DMA((2,)),
    ],
    compiler_params=pltpu.CompilerParams(
        use_tc_tiling_on_sc=False, needs_layout_passes=False
    ),
)
def _vq_decode(tokens_hbm, embed_hbm, out_hbm, m0_v, m1_v, rows_v, outt_v,
               gsem, osem):
    cid = lax.axis_index("c")
    sid = lax.axis_index("s")
    wid = sid * 2 + cid
    b = wid // TILES_PER_B
    j0 = (wid % TILES_PER_B) * (TOK_PER_W // 128)
    tb0 = (wid % TILES_PER_B) * (TOK_PER_W // 128)

    pltpu.sync_copy(tokens_hbm.at[pl.ds(j0, TOK_PER_W // 128), b, :], m0_v)

    iota = lax.iota(jnp.int32, L)

    @plsc.parallel_loop(0, TOK_PER_W // L, unroll=4)
    def _(g):
        jj = g // 8
        l0 = (g % 8) * L
        v = m0_v[jj, pl.ds(l0, L)]
        m0 = ((v >> 3) << 4) | (v & 7)
        m0_v[jj, pl.ds(l0, L)] = m0
        plsc.store_scatter(m1_v, [g * L + iota], m0 | 8)

    def issue_gather(p, sub):
        pltpu.async_copy(
            embed_hbm.at[m0_v.at[p, pl.ds(sub * W, W)]],
            rows_v.at[sub, 0],
            gsem.at[sub],
        )
        pltpu.async_copy(
            embed_hbm.at[m1_v.at[pl.ds((2 * p + sub) * W, W)]],
            rows_v.at[sub, 1],
            gsem.at[sub],
        )

    def wait_gather(sub):
        for h in range(2):
            pltpu.make_async_copy(
                embed_hbm.at[pl.ds(0, W), :], rows_v.at[sub, h], gsem.at[sub]
            ).wait()

    def wait_write(oi):
        pltpu.make_async_copy(
            outt_v.at[oi].at[:, :, pl.ds(0, 128)],
            out_hbm.at[b, :, tb0, :, :],
            osem.at[oi],
        ).wait()

    def issue_write(p, oi):
        pltpu.async_copy(
            outt_v.at[oi].at[:, :, pl.ds(0, 128)],
            out_hbm.at[b, :, tb0 + p, :, :],
            osem.at[oi],
        )

    def transpose(sub, outt):
        tcol = sub * W

        @plsc.parallel_loop(0, W, unroll=8)
        def _(t):
            t_vec = jnp.full((L,), tcol, jnp.int32) + t
            for h in range(2):
                for db in range(8):
                    d_vec = iota + (h * 128 + db * L)
                    vals = rows_v[sub, h, t, pl.ds(db * L, L)]
                    plsc.store_scatter(
                        outt, [d_vec >> 3, d_vec & 7, t_vec], vals
                    )

    issue_gather(0, 0)
    issue_gather(0, 1)

    def pair_body(p, _):
        oi = p % 2
        outt = outt_v.at[oi]
        @pl.when(p >= 2)
        def _():
            wait_write(oi)

        wait_gather(0)
        transpose(0, outt)
        issue_gather(p + 1, 0)
        wait_gather(1)
        transpose(1, outt)
        issue_gather(p + 1, 1)
        issue_write(p, oi)
        return 0

    lax.fori_loop(0, PAIRS - 1, pair_body, 0)

    p_last = PAIRS - 1
    oi = p_last % 2
    outt = outt_v.at[oi]
    wait_write(oi)
    wait_gather(0)
    transpose(0, outt)
    wait_gather(1)
    transpose(1, outt)
    issue_write(p_last, oi)

    wait_write(1 - oi)
    wait_write(oi)


def kernel(tokens, embed):
    tokens_in = tokens.reshape(B, T // 128, 128).transpose(1, 0, 2)
    embed_in = (
        embed.reshape(V // 8, 8, 2, 128).transpose(0, 2, 1, 3).reshape(2 * V, 128)
    )
    out5d = _vq_decode(tokens_in, embed_in)
    return out5d.transpose(0, 1, 3, 2, 4).reshape(B, D, T)
